# baseline (speedup 1.0000x reference)
import functools

import jax
import jax.numpy as jnp
from jax import lax
from jax.experimental import pallas as pl
from jax.experimental.pallas import tpu as pltpu
from jax.experimental.pallas import tpu_sc as plsc

B, N, C_IN, K = 2, 8192, 32, 16
EPS = 1e-5
D_PAD = 48
QT = 256
NT = 512
NEG_MASK = 3.0e38


NCH = 64
CHS = N // NCH
CAND_T = 4
BIGI = 2 ** 30


def _knn_body(p1_ref, p2_ref, idx_ref, d_ref):
    b = pl.program_id(0)
    q = p1_ref[0]
    p2t = p2_ref[0]
    qp = lax.dot_general(p2t, q, (((1,), (0,)), ((), ())),
                         preferred_element_type=jnp.float32)
    n2 = jnp.sum(p2t * p2t, axis=1, keepdims=True)
    d0 = (n2 - 2.0 * qp).reshape(NCH, CHS, QT)
    d_ref[...] = d0
    m = jnp.min(d0, axis=1)

    li3 = lax.broadcasted_iota(jnp.int32, (NCH, CHS, QT), 1)
    choff = lax.broadcasted_iota(jnp.int32, (NCH, QT), 0) * CHS
    cand_v, cand_i = [], []
    for t in range(CAND_T):
        d3 = d_ref[...]
        il = jnp.min(jnp.where(d3 <= m[:, None, :], li3, BIGI),
                     axis=1)
        cand_v.append(m)
        cand_i.append(il + choff)
        d3n = jnp.where(li3 == il[:, None, :], NEG_MASK, d3)
        if t < CAND_T - 1:
            d_ref[...] = d3n
        m = jnp.min(d3n, axis=1)
    bound = jnp.min(m, axis=0, keepdims=True)

    cv = jnp.concatenate(cand_v, axis=0)
    ci = jnp.concatenate(cand_i, axis=0)
    cols = []
    m16 = None
    for _ in range(K):
        m16 = jnp.min(cv, axis=0, keepdims=True)
        gi = jnp.min(jnp.where(cv <= m16, ci, BIGI), axis=0,
                     keepdims=True)
        cols.append(gi)
        cv = jnp.where(ci == gi, NEG_MASK, cv)
    idx_ref[0] = jnp.concatenate(cols, axis=0) + b * N

    ok = jnp.all(m16 < bound)

    @pl.when(jnp.logical_not(ok))
    def _fallback():
        d = (jnp.sum(p2t * p2t, axis=1, keepdims=True)
             - 2.0 * lax.dot_general(p2t, q, (((1,), (0,)), ((), ())),
                                     preferred_element_type=jnp.float32))
        ri = lax.broadcasted_iota(jnp.int32, (N, QT), 0)
        fcols = []
        fm = jnp.min(d, axis=0, keepdims=True)
        for t in range(K):
            fi = jnp.min(jnp.where(d <= fm, ri, BIGI), axis=0,
                         keepdims=True)
            fcols.append(fi)
            if t < K - 1:
                d = jnp.where(ri == fi, NEG_MASK, d)
                fm = jnp.min(d, axis=0, keepdims=True)
        idx_ref[0] = jnp.concatenate(fcols, axis=0) + b * N


def _knn_indices(pos1, pos2_t):
    grid = (B, N // QT)
    idx = pl.pallas_call(
        _knn_body,
        grid=grid,
        in_specs=[
            pl.BlockSpec((1, 3, QT), lambda b, q: (b, 0, q)),
            pl.BlockSpec((1, N, 3), lambda b, q: (b, 0, 0)),
        ],
        out_specs=pl.BlockSpec((1, K, QT), lambda b, q: (b, 0, q)),
        out_shape=jax.ShapeDtypeStruct((B, K, N), jnp.int32),
        scratch_shapes=[pltpu.VMEM((NCH, CHS, QT), jnp.float32)],
    )(pos1, pos2_t)
    return jnp.transpose(idx, (0, 2, 1))


_SC_CHUNK = 128


def _sc_gather(src, idx_flat):
    info = plsc.get_sparse_core_info()
    nw = info.num_cores * info.num_subcores
    e_total = idx_flat.shape[0]
    per_w = e_total // nw
    n_chunks = per_w // _SC_CHUNK
    mesh = plsc.VectorSubcoreMesh(core_axis_name="c", subcore_axis_name="s")

    @functools.partial(
        pl.kernel,
        mesh=mesh,
        compiler_params=pltpu.CompilerParams(use_tc_tiling_on_sc=False),
        out_type=jax.ShapeDtypeStruct((e_total, D_PAD), jnp.float32),
        scratch_types=[
            pltpu.VMEM((_SC_CHUNK,), jnp.int32),
            pltpu.VMEM((_SC_CHUNK, D_PAD), jnp.float32),
            pltpu.SemaphoreType.DMA,
        ],
    )
    def gk(src_hbm, idx_hbm, out_hbm, idx_v, rows_v, sem):
        wid = lax.axis_index("s") * info.num_cores + lax.axis_index("c")
        base = wid * per_w

        def body(j, carry):
            off = base + j * _SC_CHUNK
            pltpu.sync_copy(idx_hbm.at[pl.ds(off, _SC_CHUNK)], idx_v)
            pltpu.async_copy(src_hbm.at[idx_v], rows_v, sem).wait()
            pltpu.sync_copy(rows_v, out_hbm.at[pl.ds(off, _SC_CHUNK)])
            return carry

        lax.fori_loop(0, n_chunks, body, 0)

    return gk(src, idx_flat)


def _mlp_body(g_ref, xc_ref, wg_ref, wc_ref, w1_ref, s0_ref, b0_ref,
              s1_ref, b1_ref, out_ref):
    x = g_ref[0]
    h = lax.dot_general(x, wg_ref[...], (((1,), (0,)), ((), ())),
                        preferred_element_type=jnp.float32)
    c1 = lax.dot_general(xc_ref[0], wc_ref[...], (((1,), (0,)), ((), ())),
                         preferred_element_type=jnp.float32)
    h = h.reshape(NT, K, 64) + c1[:, None, :]
    h = jnp.maximum(h * s0_ref[...][None] + b0_ref[...][None], 0.0)
    h2 = lax.dot_general(h.reshape(NT * K, 64), w1_ref[...],
                         (((1,), (0,)), ((), ())),
                         preferred_element_type=jnp.float32)
    h2 = h2.reshape(NT, K, 64)
    h2 = jnp.maximum(h2 * s1_ref[...][None] + b1_ref[...][None], 0.0)
    out_ref[0] = jnp.max(h2, axis=1)


def _mlp_maxpool(g, xc, wg, wc, w1t, s0, b0, s1, b1):
    grid = (B, N // NT)
    full = lambda *s: pl.BlockSpec(s, lambda b, q: tuple(0 for _ in s))
    return pl.pallas_call(
        _mlp_body,
        grid=grid,
        in_specs=[
            pl.BlockSpec((1, NT * K, D_PAD), lambda b, q: (b, q, 0)),
            pl.BlockSpec((1, NT, 35), lambda b, q: (b, q, 0)),
            full(D_PAD, 64),
            full(35, 64),
            full(64, 64),
            full(1, 64), full(1, 64), full(1, 64), full(1, 64),
        ],
        out_specs=pl.BlockSpec((1, NT, 64), lambda b, q: (b, q, 0)),
        out_shape=jax.ShapeDtypeStruct((B, N, 64), jnp.float32),
    )(g, xc, wg, wc, w1t, s0, b0, s1, b1)


def kernel(pos1, pos2, feature1, feature2, W0, gamma0, beta0, W1, gamma1,
           beta1):
    pos1_t = jnp.transpose(pos1, (0, 2, 1))
    pos2_t = jnp.transpose(pos2, (0, 2, 1))
    feat1_t = jnp.transpose(feature1, (0, 2, 1))
    feat2_t = jnp.transpose(feature2, (0, 2, 1))

    idx = _knn_indices(pos1, pos2_t)

    src = jnp.concatenate(
        [pos2_t, feat2_t, jnp.zeros((B, N, D_PAD - 35), jnp.float32)],
        axis=-1).reshape(B * N, D_PAD)
    g = _sc_gather(src, idx.reshape(-1))

    inv = 1.0 / jnp.sqrt(1.0 + EPS)
    s0 = (gamma0 * inv)[None, :]
    b0 = beta0[None, :]
    s1 = (gamma1 * inv)[None, :]
    b1 = beta1[None, :]
    wg = jnp.concatenate(
        [W0[:, :35].T, jnp.zeros((D_PAD - 35, 64), jnp.float32)], axis=0)
    wc = jnp.concatenate([-W0[:, :3].T, W0[:, 35:].T], axis=0)
    xc = jnp.concatenate([pos1_t, feat1_t], axis=-1)

    out = _mlp_maxpool(g.reshape(B, N * K, D_PAD), xc, wg, wc, W1.T,
                       s0, b0, s1, b1)
    return (pos1, jnp.transpose(out, (0, 2, 1)))


# speedup vs baseline: 1.8620x; 1.8620x over previous
import functools

import jax
import jax.numpy as jnp
from jax import lax
from jax.experimental import pallas as pl
from jax.experimental.pallas import tpu as pltpu
from jax.experimental.pallas import tpu_sc as plsc

B, N, C_IN, K = 2, 8192, 32, 16
EPS = 1e-5
D_PAD = 48
QT = 256
NT = 512
NEG_MASK = 3.0e38


NCH = 64
CHS = N // NCH
CAND_T = 5
BIGI = 2 ** 30


def _knn_body(p1_ref, p2_ref, idx_ref, d_ref):
    b = pl.program_id(0)
    q = p1_ref[0]
    p2t = p2_ref[0]
    qp = lax.dot_general(p2t, q, (((1,), (0,)), ((), ())),
                         preferred_element_type=jnp.float32)
    n2 = jnp.sum(p2t * p2t, axis=1, keepdims=True)
    d0 = (n2 - 2.0 * qp).reshape(NCH, CHS, QT)
    d_ref[...] = d0
    m = jnp.min(d0, axis=1)

    li3 = lax.broadcasted_iota(jnp.int32, (NCH, CHS, QT), 1)
    choff = lax.broadcasted_iota(jnp.int32, (NCH, QT), 0) * CHS
    cand_v, cand_i = [], []
    for t in range(CAND_T):
        d3 = d_ref[...]
        il = jnp.min(jnp.where(d3 <= m[:, None, :], li3, BIGI),
                     axis=1)
        cand_v.append(m)
        cand_i.append(il + choff)
        d3n = jnp.where(li3 == il[:, None, :], NEG_MASK, d3)
        if t < CAND_T - 1:
            d_ref[...] = d3n
        m = jnp.min(d3n, axis=1)
    bound = jnp.min(m, axis=0, keepdims=True)

    cv = jnp.concatenate(cand_v, axis=0)
    ci = jnp.concatenate(cand_i, axis=0)
    cols = []
    m16 = None
    for _ in range(K):
        m16 = jnp.min(cv, axis=0, keepdims=True)
        gi = jnp.min(jnp.where(cv <= m16, ci, BIGI), axis=0,
                     keepdims=True)
        cols.append(gi)
        cv = jnp.where(ci == gi, NEG_MASK, cv)
    idx_ref[0] = jnp.concatenate(cols, axis=0) + b * N

    ok = jnp.all(m16 < bound)

    @pl.when(jnp.logical_not(ok))
    def _fallback():
        d = (jnp.sum(p2t * p2t, axis=1, keepdims=True)
             - 2.0 * lax.dot_general(p2t, q, (((1,), (0,)), ((), ())),
                                     preferred_element_type=jnp.float32))
        ri = lax.broadcasted_iota(jnp.int32, (N, QT), 0)
        fcols = []
        fm = jnp.min(d, axis=0, keepdims=True)
        for t in range(K):
            fi = jnp.min(jnp.where(d <= fm, ri, BIGI), axis=0,
                         keepdims=True)
            fcols.append(fi)
            if t < K - 1:
                d = jnp.where(ri == fi, NEG_MASK, d)
                fm = jnp.min(d, axis=0, keepdims=True)
        idx_ref[0] = jnp.concatenate(fcols, axis=0) + b * N


def _knn_indices(pos1, pos2_t):
    grid = (B, N // QT)
    idx = pl.pallas_call(
        _knn_body,
        grid=grid,
        in_specs=[
            pl.BlockSpec((1, 3, QT), lambda b, q: (b, 0, q)),
            pl.BlockSpec((1, N, 3), lambda b, q: (b, 0, 0)),
        ],
        out_specs=pl.BlockSpec((1, K, QT), lambda b, q: (b, 0, q)),
        out_shape=jax.ShapeDtypeStruct((B, K, N), jnp.int32),
        scratch_shapes=[pltpu.VMEM((NCH, CHS, QT), jnp.float32)],
    )(pos1, pos2_t)
    return jnp.transpose(idx, (0, 2, 1))


_SC_CHUNK = 128


def _sc_gather(src, idx_flat):
    info = plsc.get_sparse_core_info()
    nw = info.num_cores * info.num_subcores
    e_total = idx_flat.shape[0]
    per_w = e_total // nw
    n_chunks = per_w // _SC_CHUNK
    mesh = plsc.VectorSubcoreMesh(core_axis_name="c", subcore_axis_name="s")

    @functools.partial(
        pl.kernel,
        mesh=mesh,
        compiler_params=pltpu.CompilerParams(use_tc_tiling_on_sc=False),
        out_type=jax.ShapeDtypeStruct((e_total, D_PAD), jnp.float32),
        scratch_types=[
            pltpu.VMEM((_SC_CHUNK,), jnp.int32),
            pltpu.VMEM((_SC_CHUNK, D_PAD), jnp.float32),
            pltpu.SemaphoreType.DMA,
        ],
    )
    def gk(src_hbm, idx_hbm, out_hbm, idx_v, rows_v, sem):
        wid = lax.axis_index("s") * info.num_cores + lax.axis_index("c")
        base = wid * per_w

        def body(j, carry):
            off = base + j * _SC_CHUNK
            pltpu.sync_copy(idx_hbm.at[pl.ds(off, _SC_CHUNK)], idx_v)
            pltpu.async_copy(src_hbm.at[idx_v], rows_v, sem).wait()
            pltpu.sync_copy(rows_v, out_hbm.at[pl.ds(off, _SC_CHUNK)])
            return carry

        lax.fori_loop(0, n_chunks, body, 0)

    return gk(src, idx_flat)


def _mlp_body(g_ref, xc_ref, wg_ref, wc_ref, w1_ref, s0_ref, b0_ref,
              s1_ref, b1_ref, out_ref):
    x = g_ref[0]
    h = lax.dot_general(x, wg_ref[...], (((1,), (0,)), ((), ())),
                        preferred_element_type=jnp.float32)
    c1 = lax.dot_general(xc_ref[0], wc_ref[...], (((1,), (0,)), ((), ())),
                         preferred_element_type=jnp.float32)
    h = h.reshape(NT, K, 64) + c1[:, None, :]
    h = jnp.maximum(h * s0_ref[...][None] + b0_ref[...][None], 0.0)
    h2 = lax.dot_general(h.reshape(NT * K, 64), w1_ref[...],
                         (((1,), (0,)), ((), ())),
                         preferred_element_type=jnp.float32)
    h2 = h2.reshape(NT, K, 64)
    h2 = jnp.maximum(h2 * s1_ref[...][None] + b1_ref[...][None], 0.0)
    out_ref[0] = jnp.max(h2, axis=1)


def _mlp_maxpool(g, xc, wg, wc, w1t, s0, b0, s1, b1):
    grid = (B, N // NT)
    full = lambda *s: pl.BlockSpec(s, lambda b, q: tuple(0 for _ in s))
    return pl.pallas_call(
        _mlp_body,
        grid=grid,
        in_specs=[
            pl.BlockSpec((1, NT * K, D_PAD), lambda b, q: (b, q, 0)),
            pl.BlockSpec((1, NT, 35), lambda b, q: (b, q, 0)),
            full(D_PAD, 64),
            full(35, 64),
            full(64, 64),
            full(1, 64), full(1, 64), full(1, 64), full(1, 64),
        ],
        out_specs=pl.BlockSpec((1, NT, 64), lambda b, q: (b, q, 0)),
        out_shape=jax.ShapeDtypeStruct((B, N, 64), jnp.float32),
    )(g, xc, wg, wc, w1t, s0, b0, s1, b1)


def kernel(pos1, pos2, feature1, feature2, W0, gamma0, beta0, W1, gamma1,
           beta1):
    pos1_t = jnp.transpose(pos1, (0, 2, 1))
    pos2_t = jnp.transpose(pos2, (0, 2, 1))
    feat1_t = jnp.transpose(feature1, (0, 2, 1))
    feat2_t = jnp.transpose(feature2, (0, 2, 1))

    idx = _knn_indices(pos1, pos2_t)

    src = jnp.concatenate(
        [pos2_t, feat2_t, jnp.zeros((B, N, D_PAD - 35), jnp.float32)],
        axis=-1).reshape(B * N, D_PAD)
    g = _sc_gather(src, idx.reshape(-1))

    inv = 1.0 / jnp.sqrt(1.0 + EPS)
    s0 = (gamma0 * inv)[None, :]
    b0 = beta0[None, :]
    s1 = (gamma1 * inv)[None, :]
    b1 = beta1[None, :]
    wg = jnp.concatenate(
        [W0[:, :35].T, jnp.zeros((D_PAD - 35, 64), jnp.float32)], axis=0)
    wc = jnp.concatenate([-W0[:, :3].T, W0[:, 35:].T], axis=0)
    xc = jnp.concatenate([pos1_t, feat1_t], axis=-1)

    out = _mlp_maxpool(g.reshape(B, N * K, D_PAD), xc, wg, wc, W1.T,
                       s0, b0, s1, b1)
    return (pos1, jnp.transpose(out, (0, 2, 1)))


# speedup vs baseline: 1.9655x; 1.0556x over previous
import functools

import jax
import jax.numpy as jnp
from jax import lax
from jax.experimental import pallas as pl
from jax.experimental.pallas import tpu as pltpu
from jax.experimental.pallas import tpu_sc as plsc

B, N, C_IN, K = 2, 8192, 32, 16
EPS = 1e-5
D_PAD = 48
QT = 256
NT = 512
NEG_MASK = 3.0e38


NCH = 64
CHS = N // NCH
CAND_T = 5
BIGI = 2 ** 30


def _knn_body(p1_ref, p2_ref, idx_ref, d_ref):
    b = pl.program_id(0)
    q = p1_ref[0]
    p2t = p2_ref[0]
    qp = lax.dot_general(p2t, q, (((1,), (0,)), ((), ())),
                         preferred_element_type=jnp.float32)
    n2 = jnp.sum(p2t * p2t, axis=1, keepdims=True)
    d0 = (n2 - 2.0 * qp).reshape(NCH, CHS, QT)
    d_ref[...] = d0
    m = jnp.min(d0, axis=1)

    li3 = lax.broadcasted_iota(jnp.int32, (NCH, CHS, QT), 1)
    choff = lax.broadcasted_iota(jnp.int32, (NCH, QT), 0) * CHS
    cand_v, cand_i = [], []
    for t in range(CAND_T):
        d3 = d_ref[...]
        il = jnp.min(jnp.where(d3 <= m[:, None, :], li3, BIGI),
                     axis=1)
        cand_v.append(m)
        cand_i.append(il + choff)
        d3n = jnp.where(li3 == il[:, None, :], NEG_MASK, d3)
        if t < CAND_T - 1:
            d_ref[...] = d3n
        m = jnp.min(d3n, axis=1)
    bound = jnp.min(m, axis=0, keepdims=True)

    cv = jnp.concatenate(cand_v, axis=0)
    ci = jnp.concatenate(cand_i, axis=0)
    cols = []
    m16 = None
    for _ in range(K):
        m16 = jnp.min(cv, axis=0, keepdims=True)
        gi = jnp.min(jnp.where(cv <= m16, ci, BIGI), axis=0,
                     keepdims=True)
        cols.append(gi)
        cv = jnp.where(ci == gi, NEG_MASK, cv)
    idx_ref[0] = jnp.concatenate(cols, axis=0) + b * N

    ok = jnp.all(m16 < bound)

    @pl.when(jnp.logical_not(ok))
    def _fallback():
        d = (jnp.sum(p2t * p2t, axis=1, keepdims=True)
             - 2.0 * lax.dot_general(p2t, q, (((1,), (0,)), ((), ())),
                                     preferred_element_type=jnp.float32))
        ri = lax.broadcasted_iota(jnp.int32, (N, QT), 0)
        fcols = []
        fm = jnp.min(d, axis=0, keepdims=True)
        for t in range(K):
            fi = jnp.min(jnp.where(d <= fm, ri, BIGI), axis=0,
                         keepdims=True)
            fcols.append(fi)
            if t < K - 1:
                d = jnp.where(ri == fi, NEG_MASK, d)
                fm = jnp.min(d, axis=0, keepdims=True)
        idx_ref[0] = jnp.concatenate(fcols, axis=0) + b * N


def _knn_indices(pos1, pos2_t):
    grid = (B, N // QT)
    idx = pl.pallas_call(
        _knn_body,
        grid=grid,
        in_specs=[
            pl.BlockSpec((1, 3, QT), lambda b, q: (b, 0, q)),
            pl.BlockSpec((1, N, 3), lambda b, q: (b, 0, 0)),
        ],
        out_specs=pl.BlockSpec((1, K, QT), lambda b, q: (b, 0, q)),
        out_shape=jax.ShapeDtypeStruct((B, K, N), jnp.int32),
        scratch_shapes=[pltpu.VMEM((NCH, CHS, QT), jnp.float32)],
    )(pos1, pos2_t)
    return jnp.transpose(idx, (0, 2, 1))


_SC_CHUNK = 128


_SC_GRP = 8


def _sc_gather(src, idx_flat):
    info = plsc.get_sparse_core_info()
    nw = info.num_cores * info.num_subcores
    e_total = idx_flat.shape[0]
    per_w = e_total // nw
    n_chunks = per_w // _SC_CHUNK
    n_grp = n_chunks // _SC_GRP
    grp_rows = _SC_GRP * _SC_CHUNK
    mesh = plsc.VectorSubcoreMesh(core_axis_name="c", subcore_axis_name="s")
    idx2 = idx_flat.reshape(e_total // _SC_CHUNK, _SC_CHUNK)

    @functools.partial(
        pl.kernel,
        mesh=mesh,
        compiler_params=pltpu.CompilerParams(use_tc_tiling_on_sc=False),
        out_type=jax.ShapeDtypeStruct((e_total, D_PAD), jnp.float32),
        scratch_types=[
            pltpu.VMEM((n_chunks, _SC_CHUNK), jnp.int32),
            pltpu.VMEM((grp_rows, D_PAD), jnp.float32),
            pltpu.SemaphoreType.DMA,
        ],
    )
    def gk(src_hbm, idx_hbm, out_hbm, idx_v, rows_v, sem):
        wid = lax.axis_index("s") * info.num_cores + lax.axis_index("c")
        pltpu.sync_copy(idx_hbm.at[pl.ds(wid * n_chunks, n_chunks)], idx_v)
        base = wid * per_w

        def body(jg, carry):
            descs = []
            for bq in range(_SC_GRP):
                descs.append(pltpu.async_copy(
                    src_hbm.at[idx_v.at[jg * _SC_GRP + bq]],
                    rows_v.at[pl.ds(bq * _SC_CHUNK, _SC_CHUNK)], sem))
            for dsc in descs:
                dsc.wait()
            pltpu.sync_copy(rows_v,
                            out_hbm.at[pl.ds(base + jg * grp_rows, grp_rows)])
            return carry

        lax.fori_loop(0, n_grp, body, 0)

    return gk(src, idx2)


def _mlp_body(g_ref, xc_ref, wg_ref, wc_ref, w1_ref, s0_ref, b0_ref,
              s1_ref, b1_ref, out_ref):
    x = g_ref[0]
    h = lax.dot_general(x, wg_ref[...], (((1,), (0,)), ((), ())),
                        preferred_element_type=jnp.float32)
    c1 = lax.dot_general(xc_ref[0], wc_ref[...], (((1,), (0,)), ((), ())),
                         preferred_element_type=jnp.float32)
    h = h.reshape(NT, K, 64) + c1[:, None, :]
    h = jnp.maximum(h * s0_ref[...][None] + b0_ref[...][None], 0.0)
    h2 = lax.dot_general(h.reshape(NT * K, 64), w1_ref[...],
                         (((1,), (0,)), ((), ())),
                         preferred_element_type=jnp.float32)
    h2 = h2.reshape(NT, K, 64)
    h2 = jnp.maximum(h2 * s1_ref[...][None] + b1_ref[...][None], 0.0)
    out_ref[0] = jnp.max(h2, axis=1)


def _mlp_maxpool(g, xc, wg, wc, w1t, s0, b0, s1, b1):
    grid = (B, N // NT)
    full = lambda *s: pl.BlockSpec(s, lambda b, q: tuple(0 for _ in s))
    return pl.pallas_call(
        _mlp_body,
        grid=grid,
        in_specs=[
            pl.BlockSpec((1, NT * K, D_PAD), lambda b, q: (b, q, 0)),
            pl.BlockSpec((1, NT, 35), lambda b, q: (b, q, 0)),
            full(D_PAD, 64),
            full(35, 64),
            full(64, 64),
            full(1, 64), full(1, 64), full(1, 64), full(1, 64),
        ],
        out_specs=pl.BlockSpec((1, NT, 64), lambda b, q: (b, q, 0)),
        out_shape=jax.ShapeDtypeStruct((B, N, 64), jnp.float32),
    )(g, xc, wg, wc, w1t, s0, b0, s1, b1)


def kernel(pos1, pos2, feature1, feature2, W0, gamma0, beta0, W1, gamma1,
           beta1):
    pos1_t = jnp.transpose(pos1, (0, 2, 1))
    pos2_t = jnp.transpose(pos2, (0, 2, 1))
    feat1_t = jnp.transpose(feature1, (0, 2, 1))
    feat2_t = jnp.transpose(feature2, (0, 2, 1))

    idx = _knn_indices(pos1, pos2_t)

    src = jnp.concatenate(
        [pos2_t, feat2_t, jnp.zeros((B, N, D_PAD - 35), jnp.float32)],
        axis=-1).reshape(B * N, D_PAD)
    g = _sc_gather(src, idx.reshape(-1))

    inv = 1.0 / jnp.sqrt(1.0 + EPS)
    s0 = (gamma0 * inv)[None, :]
    b0 = beta0[None, :]
    s1 = (gamma1 * inv)[None, :]
    b1 = beta1[None, :]
    wg = jnp.concatenate(
        [W0[:, :35].T, jnp.zeros((D_PAD - 35, 64), jnp.float32)], axis=0)
    wc = jnp.concatenate([-W0[:, :3].T, W0[:, 35:].T], axis=0)
    xc = jnp.concatenate([pos1_t, feat1_t], axis=-1)

    out = _mlp_maxpool(g.reshape(B, N * K, D_PAD), xc, wg, wc, W1.T,
                       s0, b0, s1, b1)
    return (pos1, jnp.transpose(out, (0, 2, 1)))


# speedup vs baseline: 1.9717x; 1.0032x over previous
import functools

import jax
import jax.numpy as jnp
from jax import lax
from jax.experimental import pallas as pl
from jax.experimental.pallas import tpu as pltpu
from jax.experimental.pallas import tpu_sc as plsc

B, N, C_IN, K = 2, 8192, 32, 16
EPS = 1e-5
D_PAD = 48
QT = 256
NT = 512
NEG_MASK = 3.0e38


NCH = 64
CHS = N // NCH
CAND_T = 5
BIGI = 2 ** 30


def _knn_body(p1_ref, p2_ref, idx_ref, d_ref):
    b = pl.program_id(0)
    q = p1_ref[0]
    p2t = p2_ref[0]
    qp = lax.dot_general(p2t, q, (((1,), (0,)), ((), ())),
                         preferred_element_type=jnp.float32)
    n2 = jnp.sum(p2t * p2t, axis=1, keepdims=True)
    d0 = (n2 - 2.0 * qp).reshape(NCH, CHS, QT)
    d_ref[...] = d0
    m = jnp.min(d0, axis=1)

    li3 = lax.broadcasted_iota(jnp.int32, (NCH, CHS, QT), 1)
    choff = lax.broadcasted_iota(jnp.int32, (NCH, QT), 0) * CHS
    cand_v, cand_i = [], []
    for t in range(CAND_T):
        d3 = d_ref[...]
        il = jnp.min(jnp.where(d3 <= m[:, None, :], li3, BIGI),
                     axis=1)
        cand_v.append(m)
        cand_i.append(il + choff)
        d3n = jnp.where(li3 == il[:, None, :], NEG_MASK, d3)
        if t < CAND_T - 1:
            d_ref[...] = d3n
        m = jnp.min(d3n, axis=1)
    bound = jnp.min(m, axis=0, keepdims=True)

    cv = jnp.concatenate(cand_v, axis=0)
    ci = jnp.concatenate(cand_i, axis=0)
    cols = []
    m16 = None
    for _ in range(K):
        m16 = jnp.min(cv, axis=0, keepdims=True)
        gi = jnp.min(jnp.where(cv <= m16, ci, BIGI), axis=0,
                     keepdims=True)
        cols.append(gi)
        cv = jnp.where(ci == gi, NEG_MASK, cv)
    idx_ref[0] = jnp.concatenate(cols, axis=0) + b * N

    ok = jnp.all(m16 < bound)

    @pl.when(jnp.logical_not(ok))
    def _fallback():
        d = (jnp.sum(p2t * p2t, axis=1, keepdims=True)
             - 2.0 * lax.dot_general(p2t, q, (((1,), (0,)), ((), ())),
                                     preferred_element_type=jnp.float32))
        ri = lax.broadcasted_iota(jnp.int32, (N, QT), 0)
        fcols = []
        fm = jnp.min(d, axis=0, keepdims=True)
        for t in range(K):
            fi = jnp.min(jnp.where(d <= fm, ri, BIGI), axis=0,
                         keepdims=True)
            fcols.append(fi)
            if t < K - 1:
                d = jnp.where(ri == fi, NEG_MASK, d)
                fm = jnp.min(d, axis=0, keepdims=True)
        idx_ref[0] = jnp.concatenate(fcols, axis=0) + b * N


def _knn_indices(pos1, pos2_t):
    grid = (B, N // QT)
    idx = pl.pallas_call(
        _knn_body,
        grid=grid,
        in_specs=[
            pl.BlockSpec((1, 3, QT), lambda b, q: (b, 0, q)),
            pl.BlockSpec((1, N, 3), lambda b, q: (b, 0, 0)),
        ],
        out_specs=pl.BlockSpec((1, K, QT), lambda b, q: (b, 0, q)),
        out_shape=jax.ShapeDtypeStruct((B, K, N), jnp.int32),
        scratch_shapes=[pltpu.VMEM((NCH, CHS, QT), jnp.float32)],
    )(pos1, pos2_t)
    return jnp.transpose(idx, (0, 2, 1))


_SC_CHUNK = 128


_SC_GRP = 16


def _sc_gather(src, idx_flat):
    info = plsc.get_sparse_core_info()
    nw = info.num_cores * info.num_subcores
    e_total = idx_flat.shape[0]
    per_w = e_total // nw
    n_chunks = per_w // _SC_CHUNK
    n_grp = n_chunks // _SC_GRP
    grp_rows = _SC_GRP * _SC_CHUNK
    mesh = plsc.VectorSubcoreMesh(core_axis_name="c", subcore_axis_name="s")
    idx2 = idx_flat.reshape(e_total // _SC_CHUNK, _SC_CHUNK)

    @functools.partial(
        pl.kernel,
        mesh=mesh,
        compiler_params=pltpu.CompilerParams(use_tc_tiling_on_sc=False),
        out_type=jax.ShapeDtypeStruct((e_total, D_PAD), jnp.float32),
        scratch_types=[
            pltpu.VMEM((n_chunks, _SC_CHUNK), jnp.int32),
            pltpu.VMEM((grp_rows, D_PAD), jnp.float32),
            pltpu.SemaphoreType.DMA,
        ],
    )
    def gk(src_hbm, idx_hbm, out_hbm, idx_v, rows_v, sem):
        wid = lax.axis_index("s") * info.num_cores + lax.axis_index("c")
        pltpu.sync_copy(idx_hbm.at[pl.ds(wid * n_chunks, n_chunks)], idx_v)
        base = wid * per_w

        def body(jg, carry):
            descs = []
            for bq in range(_SC_GRP):
                descs.append(pltpu.async_copy(
                    src_hbm.at[idx_v.at[jg * _SC_GRP + bq]],
                    rows_v.at[pl.ds(bq * _SC_CHUNK, _SC_CHUNK)], sem))
            for dsc in descs:
                dsc.wait()
            pltpu.sync_copy(rows_v,
                            out_hbm.at[pl.ds(base + jg * grp_rows, grp_rows)])
            return carry

        lax.fori_loop(0, n_grp, body, 0)

    return gk(src, idx2)


def _mlp_body(g_ref, xc_ref, wg_ref, wc_ref, w1_ref, s0_ref, b0_ref,
              s1_ref, b1_ref, out_ref):
    x = g_ref[0]
    h = lax.dot_general(x, wg_ref[...], (((1,), (0,)), ((), ())),
                        preferred_element_type=jnp.float32)
    c1 = lax.dot_general(xc_ref[0], wc_ref[...], (((1,), (0,)), ((), ())),
                         preferred_element_type=jnp.float32)
    h = h.reshape(NT, K, 64) + c1[:, None, :]
    h = jnp.maximum(h * s0_ref[...][None] + b0_ref[...][None], 0.0)
    h2 = lax.dot_general(h.reshape(NT * K, 64), w1_ref[...],
                         (((1,), (0,)), ((), ())),
                         preferred_element_type=jnp.float32)
    h2 = h2.reshape(NT, K, 64)
    h2 = jnp.maximum(h2 * s1_ref[...][None] + b1_ref[...][None], 0.0)
    out_ref[0] = jnp.max(h2, axis=1)


def _mlp_maxpool(g, xc, wg, wc, w1t, s0, b0, s1, b1):
    grid = (B, N // NT)
    full = lambda *s: pl.BlockSpec(s, lambda b, q: tuple(0 for _ in s))
    return pl.pallas_call(
        _mlp_body,
        grid=grid,
        in_specs=[
            pl.BlockSpec((1, NT * K, D_PAD), lambda b, q: (b, q, 0)),
            pl.BlockSpec((1, NT, 35), lambda b, q: (b, q, 0)),
            full(D_PAD, 64),
            full(35, 64),
            full(64, 64),
            full(1, 64), full(1, 64), full(1, 64), full(1, 64),
        ],
        out_specs=pl.BlockSpec((1, NT, 64), lambda b, q: (b, q, 0)),
        out_shape=jax.ShapeDtypeStruct((B, N, 64), jnp.float32),
    )(g, xc, wg, wc, w1t, s0, b0, s1, b1)


def kernel(pos1, pos2, feature1, feature2, W0, gamma0, beta0, W1, gamma1,
           beta1):
    pos1_t = jnp.transpose(pos1, (0, 2, 1))
    pos2_t = jnp.transpose(pos2, (0, 2, 1))
    feat1_t = jnp.transpose(feature1, (0, 2, 1))
    feat2_t = jnp.transpose(feature2, (0, 2, 1))

    idx = _knn_indices(pos1, pos2_t)

    src = jnp.concatenate(
        [pos2_t, feat2_t, jnp.zeros((B, N, D_PAD - 35), jnp.float32)],
        axis=-1).reshape(B * N, D_PAD)
    g = _sc_gather(src, idx.reshape(-1))

    inv = 1.0 / jnp.sqrt(1.0 + EPS)
    s0 = (gamma0 * inv)[None, :]
    b0 = beta0[None, :]
    s1 = (gamma1 * inv)[None, :]
    b1 = beta1[None, :]
    wg = jnp.concatenate(
        [W0[:, :35].T, jnp.zeros((D_PAD - 35, 64), jnp.float32)], axis=0)
    wc = jnp.concatenate([-W0[:, :3].T, W0[:, 35:].T], axis=0)
    xc = jnp.concatenate([pos1_t, feat1_t], axis=-1)

    out = _mlp_maxpool(g.reshape(B, N * K, D_PAD), xc, wg, wc, W1.T,
                       s0, b0, s1, b1)
    return (pos1, jnp.transpose(out, (0, 2, 1)))
